# baseline (speedup 1.0000x reference)
import functools

import jax
import jax.numpy as jnp
from jax import lax
from jax.experimental import pallas as pl
from jax.experimental.pallas import tpu as pltpu
from jax.experimental.pallas import tpu_sc as plsc

_F = 26
_CARD = 1000000
_C = 2
_B = 16384
_FS = 1000448

_NC = 2
_NS = 16
_L = 16
_NW = _NC * _NS
_BPW = _B // _NW
_NROW = _BPW * _F


def _pack_body(w_hbm, out_hbm, av0, bv0, ov0, av1, bv1, ov1, isem, osem):
    bufs = ((av0, bv0, ov0), (av1, bv1, ov1))

    def in_copies(f, slot):
        a, b, _ = bufs[slot]
        return [
            pltpu.make_async_copy(w_hbm.at[f, 0], a, isem.at[slot, 0]),
            pltpu.make_async_copy(w_hbm.at[f, 1], b, isem.at[slot, 1]),
        ]

    def start_in(f, slot):
        for c in in_copies(f, slot):
            c.start()

    def wait_in(f, slot):
        for c in in_copies(f, slot):
            c.wait()

    def out_copy(f, slot):
        return pltpu.make_async_copy(bufs[slot][2],
                                     out_hbm.at[pl.ds(f * _FS, _FS)],
                                     osem.at[slot])

    zero_tail = jnp.zeros((_FS - _CARD,), jnp.bfloat16)
    ov0[pl.ds(_CARD, _FS - _CARD)] = zero_tail
    ov1[pl.ds(_CARD, _FS - _CARD)] = zero_tail
    start_in(0, 0)
    start_in(1, 1)

    def step(g, _):
        for slot in (0, 1):
            f = 2 * g + slot
            a, b, o = bufs[slot]

            @pl.when(g >= 1)
            def _():
                out_copy(f - 2, slot).wait()

            wait_in(f, slot)
            o[pl.ds(0, _CARD)] = (a[...] - b[...]).astype(jnp.bfloat16)
            out_copy(f, slot).start()

            @pl.when(f + 2 < _F)
            def _():
                start_in(f + 2, slot)

        return 0

    lax.fori_loop(0, _F // 2, step, 0)
    out_copy(_F - 2, 0).wait()
    out_copy(_F - 1, 1).wait()


def _pack(w_t):
    return pl.pallas_call(
        _pack_body,
        in_specs=[pl.BlockSpec(memory_space=pltpu.MemorySpace.HBM)],
        out_specs=pl.BlockSpec(memory_space=pltpu.MemorySpace.HBM),
        out_shape=jax.ShapeDtypeStruct((_F * _FS,), jnp.bfloat16),
        scratch_shapes=[
            pltpu.VMEM((_CARD,), jnp.float32),
            pltpu.VMEM((_CARD,), jnp.float32),
            pltpu.VMEM((_FS,), jnp.bfloat16),
            pltpu.VMEM((_CARD,), jnp.float32),
            pltpu.VMEM((_CARD,), jnp.float32),
            pltpu.VMEM((_FS,), jnp.bfloat16),
            pltpu.SemaphoreType.DMA((2, 2)),
            pltpu.SemaphoreType.DMA((2,)),
        ],
    )(w_t)


def _sc_body(x_hbm, w_hbm, bias_hbm, out_hbm, ids_v, idx_v, idxw_v, rows_v,
             bias_v, out_v, sem):
    wid = lax.axis_index("s") * _NC + lax.axis_index("c")
    base = wid * _BPW
    pltpu.sync_copy(x_hbm.at[pl.ds(base * _F, _NROW)], ids_v)
    pltpu.sync_copy(bias_hbm, bias_v)

    iota = lax.iota(jnp.int32, _L)
    zeros = jnp.zeros((_L,), jnp.int32)
    ones = zeros + 1

    def build(j, _):
        f = lax.shift_right_logical(j, 5)
        bbase = j * _L - f * _BPW
        b_vec = bbase + iota
        vals = plsc.load_gather(ids_v, [b_vec * _F + f])
        idx = vals + _FS * f
        idx_v[pl.ds(j * _L, _L)] = idx
        idxw_v[pl.ds(j * _L, _L)] = lax.shift_right_logical(idx, 1)
        return 0

    lax.fori_loop(0, _NROW // _L, build, 0)

    pltpu.async_copy(w_hbm.at[idxw_v], rows_v, sem).wait()

    dbias = plsc.load_gather(bias_v, [zeros]) - plsc.load_gather(bias_v, [ones])
    himask = jnp.full((_L,), -65536, jnp.int32)

    def chunk(i, _):
        off = i * _L
        acc = jnp.zeros((_L,), jnp.float32)
        for f in range(_F):
            pos = pl.ds(f * _BPW + off, _L)
            xi = plsc.bitcast(rows_v[pos], jnp.int32)
            par = lax.bitwise_and(idx_v[pos], 1)
            d = jnp.where(par == 1,
                          plsc.bitcast(lax.bitwise_and(xi, himask),
                                       jnp.float32),
                          plsc.bitcast(lax.shift_left(xi, 16), jnp.float32))
            acc = acc + d
        x = acc + dbias
        p0 = 1.0 / (1.0 + jnp.exp(-x))
        p1 = 1.0 / (1.0 + jnp.exp(x))
        e2 = (off + iota) * 2
        plsc.store_scatter(out_v, [e2], p0)
        plsc.store_scatter(out_v, [e2 + 1], p1)
        return 0

    lax.fori_loop(0, _BPW // _L, chunk, 0)

    pltpu.sync_copy(out_v, out_hbm.at[pl.ds(base * _C, _BPW * _C)])


@functools.partial(
    pl.kernel,
    compiler_params=pltpu.CompilerParams(use_tc_tiling_on_sc=False,
                                         needs_layout_passes=False),
    out_type=jax.ShapeDtypeStruct((_B * _C,), jnp.float32),
    mesh=plsc.VectorSubcoreMesh(core_axis_name="c", subcore_axis_name="s",
                                num_cores=_NC, num_subcores=_NS),
    scratch_types=[
        pltpu.VMEM((_NROW,), jnp.int32),
        pltpu.VMEM((_NROW,), jnp.int32),
        pltpu.VMEM((_NROW,), jnp.int32),
        pltpu.VMEM((_NROW,), jnp.float32),
        pltpu.VMEM((_L,), jnp.float32),
        pltpu.VMEM((_BPW * _C,), jnp.float32),
        pltpu.SemaphoreType.DMA,
    ],
)
def _wide_linear_sc(x_hbm, w_hbm, bias_hbm, out_hbm, *scratch):
    _sc_body(x_hbm, w_hbm, bias_hbm, out_hbm, *scratch)


def kernel(x_ids, W, bias):
    x32 = x_ids.astype(jnp.int32).reshape(_B * _F)
    w_t = jnp.transpose(W, (0, 2, 1))
    packed = _pack(w_t)
    packed_w = lax.bitcast_convert_type(
        packed.reshape(_F * _FS // 2, 2), jnp.float32)
    bias16 = jnp.zeros((_L,), jnp.float32).at[:_C].set(bias.astype(jnp.float32))
    out = _wide_linear_sc(x32, packed_w, bias16)
    return out.reshape(_B, _C)


# speedup vs baseline: 36.6409x; 36.6409x over previous
import functools

import jax
import jax.numpy as jnp
from jax import lax
from jax.experimental import pallas as pl
from jax.experimental.pallas import tpu as pltpu
from jax.experimental.pallas import tpu_sc as plsc

_F = 26
_CARD = 1000000
_C = 2
_B = 16384
_H = 499968
_TB = 999872
_KW = 500096

_NC = 2
_NS = 16
_L = 16
_NW = _NC * _NS
_BPW = _B // _NW
_NROW = _BPW * _F


def _pack_body(w_hbm, out_hbm, av0, bv0, ov0, av1, bv1, ov1, isem, osem):
    bufs = ((av0, bv0, ov0), (av1, bv1, ov1))

    def in_copies(f, slot):
        a, b, _ = bufs[slot]
        return [
            pltpu.make_async_copy(w_hbm.at[f, 0], a, isem.at[slot, 0]),
            pltpu.make_async_copy(w_hbm.at[f, 1], b, isem.at[slot, 1]),
        ]

    def start_in(f, slot):
        for c in in_copies(f, slot):
            c.start()

    def wait_in(f, slot):
        for c in in_copies(f, slot):
            c.wait()

    def out_copy(f, slot):
        return pltpu.make_async_copy(bufs[slot][2],
                                     out_hbm.at[pl.ds(f * _KW, _KW)],
                                     osem.at[slot])

    def bf16_bits(x):
        return lax.bitcast_convert_type(x, jnp.int32) + 0x8000

    start_in(0, 0)
    start_in(1, 1)

    def step(g, _):
        for slot in (0, 1):
            f = 2 * g + slot
            a, b, o = bufs[slot]

            @pl.when(g >= 1)
            def _():
                out_copy(f - 2, slot).wait()

            wait_in(f, slot)
            for t in range(2):
                o0 = t * (_H // 2)
                ilo = bf16_bits(a[pl.ds(o0, _H // 2)]
                                - b[pl.ds(o0, _H // 2)])
                ihi = bf16_bits(a[pl.ds(_H + o0, _H // 2)]
                                - b[pl.ds(_H + o0, _H // 2)])
                w = lax.bitwise_or(
                    lax.shift_right_logical(ilo, 16),
                    lax.bitwise_and(ihi, jnp.int32(-65536)))
                o[pl.ds(o0, _H // 2)] = lax.bitcast_convert_type(
                    w, jnp.float32)
            it = bf16_bits(a[pl.ds(_TB, 128)] - b[pl.ds(_TB, 128)])
            o[pl.ds(_H, 128)] = lax.bitcast_convert_type(
                lax.shift_right_logical(it, 16), jnp.float32)
            out_copy(f, slot).start()

            @pl.when(f + 2 < _F)
            def _():
                start_in(f + 2, slot)

        return 0

    lax.fori_loop(0, _F // 2, step, 0)
    out_copy(_F - 2, 0).wait()
    out_copy(_F - 1, 1).wait()


def _pack(w_t):
    return pl.pallas_call(
        _pack_body,
        compiler_params=pltpu.CompilerParams(
            vmem_limit_bytes=100 * 1024 * 1024),
        in_specs=[pl.BlockSpec(memory_space=pltpu.MemorySpace.HBM)],
        out_specs=pl.BlockSpec(memory_space=pltpu.MemorySpace.HBM),
        out_shape=jax.ShapeDtypeStruct((_F * _KW,), jnp.float32),
        scratch_shapes=[
            pltpu.VMEM((_CARD,), jnp.float32),
            pltpu.VMEM((_CARD,), jnp.float32),
            pltpu.VMEM((_KW,), jnp.float32),
            pltpu.VMEM((_CARD,), jnp.float32),
            pltpu.VMEM((_CARD,), jnp.float32),
            pltpu.VMEM((_KW,), jnp.float32),
            pltpu.SemaphoreType.DMA((2, 2)),
            pltpu.SemaphoreType.DMA((2,)),
        ],
    )(w_t)


def _sc_body(x_hbm, w_hbm, bias_hbm, out_hbm, ids_v, idx_v, idxw_v, rows_v,
             bias_v, out_v, sem):
    wid = lax.axis_index("s") * _NC + lax.axis_index("c")
    base = wid * _BPW
    pltpu.sync_copy(x_hbm.at[pl.ds(base * _F, _NROW)], ids_v)
    pltpu.sync_copy(bias_hbm, bias_v)

    iota = lax.iota(jnp.int32, _L)
    zeros = jnp.zeros((_L,), jnp.int32)
    ones = zeros + 1

    def build(j, _):
        f = lax.shift_right_logical(j, 5)
        bbase = j * _L - f * _BPW
        b_vec = bbase + iota
        r = plsc.load_gather(ids_v, [b_vec * _F + f])
        in_hi = jnp.logical_and(r >= _H, r < _TB + 64)
        m = jnp.where(r >= _TB + 64, r - _TB + _H,
                      jnp.where(r >= _H, r - _H, r))
        idx_v[pl.ds(j * _L, _L)] = jnp.where(in_hi, ones, zeros)
        idxw_v[pl.ds(j * _L, _L)] = m + _KW * f
        return 0

    lax.fori_loop(0, _NROW // _L, build, 0)

    pltpu.async_copy(w_hbm.at[idxw_v], rows_v, sem).wait()

    dbias = plsc.load_gather(bias_v, [zeros]) - plsc.load_gather(bias_v, [ones])
    himask = jnp.full((_L,), -65536, jnp.int32)

    def chunk(i, _):
        off = i * _L
        acc = jnp.zeros((_L,), jnp.float32)
        for f in range(_F):
            pos = pl.ds(f * _BPW + off, _L)
            xi = plsc.bitcast(rows_v[pos], jnp.int32)
            d = jnp.where(idx_v[pos] == 1,
                          plsc.bitcast(lax.bitwise_and(xi, himask),
                                       jnp.float32),
                          plsc.bitcast(lax.shift_left(xi, 16), jnp.float32))
            acc = acc + d
        x = acc + dbias
        p0 = 1.0 / (1.0 + jnp.exp(-x))
        p1 = 1.0 / (1.0 + jnp.exp(x))
        e2 = (off + iota) * 2
        plsc.store_scatter(out_v, [e2], p0)
        plsc.store_scatter(out_v, [e2 + 1], p1)
        return 0

    lax.fori_loop(0, _BPW // _L, chunk, 0)

    pltpu.sync_copy(out_v, out_hbm.at[pl.ds(base * _C, _BPW * _C)])


@functools.partial(
    pl.kernel,
    compiler_params=pltpu.CompilerParams(use_tc_tiling_on_sc=False,
                                         needs_layout_passes=False),
    out_type=jax.ShapeDtypeStruct((_B * _C,), jnp.float32),
    mesh=plsc.VectorSubcoreMesh(core_axis_name="c", subcore_axis_name="s",
                                num_cores=_NC, num_subcores=_NS),
    scratch_types=[
        pltpu.VMEM((_NROW,), jnp.int32),
        pltpu.VMEM((_NROW,), jnp.int32),
        pltpu.VMEM((_NROW,), jnp.int32),
        pltpu.VMEM((_NROW,), jnp.float32),
        pltpu.VMEM((_L,), jnp.float32),
        pltpu.VMEM((_BPW * _C,), jnp.float32),
        pltpu.SemaphoreType.DMA,
    ],
)
def _wide_linear_sc(x_hbm, w_hbm, bias_hbm, out_hbm, *scratch):
    _sc_body(x_hbm, w_hbm, bias_hbm, out_hbm, *scratch)


def kernel(x_ids, W, bias):
    x32 = x_ids.astype(jnp.int32).reshape(_B * _F)
    w_t = jnp.transpose(W, (0, 2, 1))
    packed = _pack(w_t)
    bias16 = jnp.zeros((_L,), jnp.float32).at[:_C].set(bias.astype(jnp.float32))
    out = _wide_linear_sc(x32, packed, bias16)
    return out.reshape(_B, _C)
